# baseline (speedup 1.0000x reference)
import jax
import jax.numpy as jnp
from jax import lax
from jax.experimental import pallas as pl
from jax.experimental.pallas import tpu as pltpu
from jax.experimental.pallas import tpu_sc as plsc

_B, _T, _H, _W, _D = 8, 8, 14, 14, 768
_HW = _H * _W
_L = _T * _HW
_EPS = 1e-06
_NC, _NS = 2, 16
_NWORK = _NC * _NS
_RPW = _L // _NWORK
_CH = _RPW * _D


def _pos_kernel(te_ref, ve_ref, he_ref, g_ref, b_ref, o_ref):
    def onehot(idx_fn, n):
        row = lax.broadcasted_iota(jnp.int32, (_L, n), 0)
        col = lax.broadcasted_iota(jnp.int32, (_L, n), 1)
        return (idx_fn(row) == col).astype(jnp.float32)

    pos = (
        lax.dot(onehot(lambda r: r // _HW, _T), te_ref[:],
                preferred_element_type=jnp.float32)
        + lax.dot(onehot(lambda r: (r // _W) % _H, _H), ve_ref[:],
                  preferred_element_type=jnp.float32)
        + lax.dot(onehot(lambda r: r % _W, _W), he_ref[:],
                  preferred_element_type=jnp.float32)
    )
    mean = jnp.mean(pos, axis=-1, keepdims=True)
    c = pos - mean
    var = jnp.mean(c * c, axis=-1, keepdims=True)
    o_ref[:] = c * lax.rsqrt(var + _EPS) * g_ref[:] + b_ref[:]


def _sc_add_kernel(x_ref, pos_ref, o_ref, pos_v, xa, xb,
                   sa_in, sb_in, sa_out, sb_out):
    c = lax.axis_index("c")
    s = lax.axis_index("s")
    w = s * _NC + c
    base = w * _CH

    pltpu.sync_copy(pos_ref.at[pl.ds(base, _CH)], pos_v)

    bufs = (xa, xb)
    sins = (sa_in, sb_in)
    souts = (sa_out, sb_out)
    in_h = [None, None]
    out_h = [None, None]
    in_h[0] = pltpu.async_copy(x_ref.at[0, pl.ds(base, _CH)], xa, sa_in)
    for b in range(_B):
        cur = b & 1
        nxt = cur ^ 1
        if b + 1 < _B:
            if out_h[nxt] is not None:
                out_h[nxt].wait()
            in_h[nxt] = pltpu.async_copy(
                x_ref.at[b + 1, pl.ds(base, _CH)], bufs[nxt], sins[nxt])
        in_h[cur].wait()
        buf = bufs[cur]

        @plsc.parallel_loop(0, _CH, step=16)
        def _add(i):
            plsc.addupdate(buf.at[pl.ds(i, 16)], pos_v[pl.ds(i, 16)])

        out_h[cur] = pltpu.async_copy(
            buf, o_ref.at[b, pl.ds(base, _CH)], souts[cur])
    out_h[0].wait()
    out_h[1].wait()


def kernel(inputs, temporal_emb, vertical_emb, horizontal_emb, gamma, beta,
           dimensions):
    pos_ln = pl.pallas_call(
        _pos_kernel,
        out_shape=jax.ShapeDtypeStruct((_L, _D), jnp.float32),
    )(temporal_emb, vertical_emb, horizontal_emb,
      gamma.reshape(1, _D), beta.reshape(1, _D))

    x2 = inputs.reshape(_B, _L * _D)
    sc_add = pl.kernel(
        _sc_add_kernel,
        out_type=jax.ShapeDtypeStruct((_B, _L * _D), jnp.float32),
        mesh=plsc.VectorSubcoreMesh(core_axis_name="c", subcore_axis_name="s"),
        scratch_types=[
            pltpu.VMEM((_CH,), jnp.float32),
            pltpu.VMEM((_CH,), jnp.float32),
            pltpu.VMEM((_CH,), jnp.float32),
            pltpu.SemaphoreType.DMA,
            pltpu.SemaphoreType.DMA,
            pltpu.SemaphoreType.DMA,
            pltpu.SemaphoreType.DMA,
        ],
    )
    out2 = sc_add(x2, pos_ln.reshape(_L * _D))
    return out2.reshape(_B, _L, _D)


# speedup vs baseline: 1.4851x; 1.4851x over previous
import jax
import jax.numpy as jnp
from jax import lax
from jax.experimental import pallas as pl
from jax.experimental.pallas import tpu as pltpu
from jax.experimental.pallas import tpu_sc as plsc

_B, _T, _H, _W, _D = 8, 8, 14, 14, 768
_HW = _H * _W
_L = _T * _HW
_EPS = 1e-06
_NC, _NS = 2, 16
_NWORK = _NC * _NS
_RPW = _L // _NWORK
_CH = _RPW * _D


def _pos_kernel(te_ref, ve_ref, he_ref, g_ref, b_ref, o_ref):
    def onehot(idx_fn, n):
        row = lax.broadcasted_iota(jnp.int32, (_L, n), 0)
        col = lax.broadcasted_iota(jnp.int32, (_L, n), 1)
        return (idx_fn(row) == col).astype(jnp.float32)

    pos = (
        lax.dot(onehot(lambda r: r // _HW, _T), te_ref[:],
                preferred_element_type=jnp.float32)
        + lax.dot(onehot(lambda r: (r // _W) % _H, _H), ve_ref[:],
                  preferred_element_type=jnp.float32)
        + lax.dot(onehot(lambda r: r % _W, _W), he_ref[:],
                  preferred_element_type=jnp.float32)
    )
    mean = jnp.mean(pos, axis=-1, keepdims=True)
    c = pos - mean
    var = jnp.mean(c * c, axis=-1, keepdims=True)
    o_ref[:] = c * lax.rsqrt(var + _EPS) * g_ref[:] + b_ref[:]


def _sc_add_kernel(x_ref, pos_ref, o_ref, pos_v, xa, xb,
                   sa_in, sb_in, sa_out, sb_out):
    c = lax.axis_index("c")
    s = lax.axis_index("s")
    w = s * _NC + c
    base = w * _CH

    pltpu.sync_copy(pos_ref.at[pl.ds(base, _CH)], pos_v)

    bufs = (xa, xb)
    sins = (sa_in, sb_in)
    souts = (sa_out, sb_out)
    in_h = [None, None]
    out_h = [None, None]
    in_h[0] = pltpu.async_copy(x_ref.at[0, pl.ds(base, _CH)], xa, sa_in)
    for b in range(_B):
        cur = b & 1
        nxt = cur ^ 1
        if b + 1 < _B:
            if out_h[nxt] is not None:
                out_h[nxt].wait()
            in_h[nxt] = pltpu.async_copy(
                x_ref.at[b + 1, pl.ds(base, _CH)], bufs[nxt], sins[nxt])
        in_h[cur].wait()
        buf = bufs[cur]

        @plsc.parallel_loop(0, _CH, step=16, unroll=8)
        def _add(i):
            plsc.addupdate(buf.at[pl.ds(i, 16)], pos_v[pl.ds(i, 16)])

        out_h[cur] = pltpu.async_copy(
            buf, o_ref.at[b, pl.ds(base, _CH)], souts[cur])
    out_h[0].wait()
    out_h[1].wait()


def kernel(inputs, temporal_emb, vertical_emb, horizontal_emb, gamma, beta,
           dimensions):
    pos_ln = pl.pallas_call(
        _pos_kernel,
        out_shape=jax.ShapeDtypeStruct((_L, _D), jnp.float32),
    )(temporal_emb, vertical_emb, horizontal_emb,
      gamma.reshape(1, _D), beta.reshape(1, _D))

    x2 = inputs.reshape(_B, _L * _D)
    sc_add = pl.kernel(
        _sc_add_kernel,
        out_type=jax.ShapeDtypeStruct((_B, _L * _D), jnp.float32),
        mesh=plsc.VectorSubcoreMesh(core_axis_name="c", subcore_axis_name="s"),
        compiler_params=pltpu.CompilerParams(use_tc_tiling_on_sc=True),
        scratch_types=[
            pltpu.VMEM((_CH,), jnp.float32),
            pltpu.VMEM((_CH,), jnp.float32),
            pltpu.VMEM((_CH,), jnp.float32),
            pltpu.SemaphoreType.DMA,
            pltpu.SemaphoreType.DMA,
            pltpu.SemaphoreType.DMA,
            pltpu.SemaphoreType.DMA,
        ],
    )
    out2 = sc_add(x2, pos_ln.reshape(_L * _D))
    return out2.reshape(_B, _L, _D)


# speedup vs baseline: 6.5710x; 4.4246x over previous
import jax
import jax.numpy as jnp
from jax.experimental import pallas as pl
from jax.experimental.pallas import tpu as pltpu

_B, _T, _H, _W, _D = 8, 8, 14, 14, 768
_HW = _H * _W
_L = _T * _HW
_EPS = 1e-06
_BL = 1568
_NJ = _L // _BL
_BB = 2


def _fused_kernel(x_ref, te_ref, ve_ref, he_ref, g_ref, b_ref, o_ref,
                  pos_ref):
    b = pl.program_id(0)
    j = pl.program_id(1)

    @pl.when((b == 0) & (j == 0))
    def _build_pos():
        def onehot(idx_fn, n):
            row = jax.lax.broadcasted_iota(jnp.int32, (_L, n), 0)
            col = jax.lax.broadcasted_iota(jnp.int32, (_L, n), 1)
            return (idx_fn(row) == col).astype(jnp.float32)

        pos = (
            jax.lax.dot(onehot(lambda r: r // _HW, _T), te_ref[:],
                        preferred_element_type=jnp.float32)
            + jax.lax.dot(onehot(lambda r: (r // _W) % _H, _H), ve_ref[:],
                          preferred_element_type=jnp.float32)
            + jax.lax.dot(onehot(lambda r: r % _W, _W), he_ref[:],
                          preferred_element_type=jnp.float32)
        )
        mean = jnp.mean(pos, axis=-1, keepdims=True)
        c = pos - mean
        var = jnp.mean(c * c, axis=-1, keepdims=True)
        pos_ref[:] = c * jax.lax.rsqrt(var + _EPS) * g_ref[:] + b_ref[:]

    o_ref[:] = x_ref[:] + pos_ref[pl.ds(j * _BL, _BL), :][None]


def kernel(inputs, temporal_emb, vertical_emb, horizontal_emb, gamma, beta,
           dimensions):
    g = gamma.reshape(1, _D)
    be = beta.reshape(1, _D)
    out = pl.pallas_call(
        _fused_kernel,
        grid=(_B // _BB, _NJ),
        in_specs=[
            pl.BlockSpec((_BB, _BL, _D), lambda b, j: (b, j, 0)),
            pl.BlockSpec((_T, _D), lambda b, j: (0, 0)),
            pl.BlockSpec((_H, _D), lambda b, j: (0, 0)),
            pl.BlockSpec((_W, _D), lambda b, j: (0, 0)),
            pl.BlockSpec((1, _D), lambda b, j: (0, 0)),
            pl.BlockSpec((1, _D), lambda b, j: (0, 0)),
        ],
        out_specs=pl.BlockSpec((_BB, _BL, _D), lambda b, j: (b, j, 0)),
        out_shape=jax.ShapeDtypeStruct((_B, _L, _D), jnp.float32),
        scratch_shapes=[
            pltpu.VMEM((_L, _D), jnp.float32),
        ],
        compiler_params=pltpu.CompilerParams(
            dimension_semantics=("arbitrary", "arbitrary"),
        ),
    )(inputs, temporal_emb, vertical_emb, horizontal_emb, g, be)
    return out
